# baseline (speedup 1.0000x reference)
import functools

import jax
import jax.numpy as jnp
from jax import lax
from jax.experimental import pallas as pl
from jax.experimental.pallas import tpu as pltpu
from jax.experimental.pallas import tpu_sc as plsc

_K = 16
_RA = 256
_RD = 512
_PCH = 8


def _knn_mlp1_body(xblk_ref, xT_ref, xfull_ref, W1_ref, b1_ref, Wr1_ref,
                   br1_ref, Wr2_ref, br2_ref, lf_ref, idx_ref):
    b = pl.program_id(0)
    N = xT_ref.shape[2]
    xblk = xblk_ref[0]
    xT = xT_ref[0]
    G = jax.lax.dot(xblk, xT, preferred_element_type=jnp.float32)
    xx_i = jnp.sum(xblk * xblk, axis=1, keepdims=True)
    xx_j = jnp.sum(xT * xT, axis=0, keepdims=True)
    d = 2.0 * G - xx_i - xx_j
    colf = jax.lax.broadcasted_iota(jnp.int32, (_RA, N), 1).astype(jnp.float32)
    idxs = []
    for k in range(_K):
        m = jnp.max(d, axis=1, keepdims=True)
        eq = d == m
        ik = jnp.min(jnp.where(eq, colf, 2.0 * N), axis=1, keepdims=True)
        idxs.append(ik)
        if k + 1 < _K:
            d = jnp.where(eq, -jnp.inf, d)
    idx = jnp.concatenate(idxs, axis=1).astype(jnp.int32)
    xfull = xfull_ref[0]
    oh0 = (colf == idxs[0]).astype(jnp.float32)
    oh1 = (colf == idxs[1]).astype(jnp.float32)
    nb0 = jax.lax.dot(oh0, xfull, preferred_element_type=jnp.float32)
    nb1 = jax.lax.dot(oh1, xfull, preferred_element_type=jnp.float32)
    acc = jnp.zeros((_RA, 128), jnp.float32) + b1_ref[...]
    for r in range(3):
        acc += xblk[:, r:r + 1] * W1_ref[r:r + 1, :]
    for a in range(3):
        for c in range(3):
            acc += (nb0[:, a:a + 1] * nb1[:, c:c + 1]) * W1_ref[3 + 3 * a + c:4 + 3 * a + c, :]
    h1 = jnp.maximum(acc, 0.0)
    h2 = jax.lax.dot(h1, Wr1_ref[...], preferred_element_type=jnp.float32) + br1_ref[...] + h1
    h2 = jnp.maximum(h2, 0.0)
    lf = jax.lax.dot(h2, Wr2_ref[...], preferred_element_type=jnp.float32) + br2_ref[...] + h2
    lf_ref[0] = jnp.maximum(lf, 0.0)
    idx_ref[0] = idx + b * N


def _run_knn_mlp1(pts_pad, pts_padT, W1, b1, Wr1, br1, Wr2, br2):
    B, N, _ = pts_pad.shape
    grid = (B, N // _RA)
    return pl.pallas_call(
        _knn_mlp1_body,
        grid=grid,
        in_specs=[
            pl.BlockSpec((1, _RA, 128), lambda b, n: (b, n, 0)),
            pl.BlockSpec((1, 128, N), lambda b, n: (b, 0, 0)),
            pl.BlockSpec((1, N, 128), lambda b, n: (b, 0, 0)),
            pl.BlockSpec((12, 128), lambda b, n: (0, 0)),
            pl.BlockSpec((1, 128), lambda b, n: (0, 0)),
            pl.BlockSpec((128, 128), lambda b, n: (0, 0)),
            pl.BlockSpec((1, 128), lambda b, n: (0, 0)),
            pl.BlockSpec((128, 128), lambda b, n: (0, 0)),
            pl.BlockSpec((1, 128), lambda b, n: (0, 0)),
        ],
        out_specs=[
            pl.BlockSpec((1, _RA, 128), lambda b, n: (b, n, 0)),
            pl.BlockSpec((1, _RA, _K), lambda b, n: (b, n, 0)),
        ],
        out_shape=[
            jax.ShapeDtypeStruct((B, N, 128), jnp.float32),
            jax.ShapeDtypeStruct((B, N, _K), jnp.int32),
        ],
    )(pts_pad, pts_padT, pts_pad, W1, b1, Wr1, br1, Wr2, br2)


def _make_pool(M, C, packed):
    info = plsc.get_sparse_core_info()
    NC, NS = info.num_cores, info.num_subcores
    NW = NC * NS
    ppw = M // NW
    nit = ppw // _PCH
    mesh = plsc.VectorSubcoreMesh(core_axis_name="c", subcore_axis_name="s")

    C2 = C // 2 if packed else C
    dt = jnp.int32 if packed else jnp.float32

    @functools.partial(
        pl.kernel, mesh=mesh,
        out_type=jax.ShapeDtypeStruct((M, C2), dt),
        scratch_types=[
            pltpu.VMEM((2, _PCH * _K), jnp.int32),
            pltpu.VMEM((2, _PCH * _K, C2), dt),
            pltpu.VMEM((_PCH, C2), dt),
            pltpu.SemaphoreType.DMA((2,)),
        ],
    )
    def pool(tbl_hbm, idx_hbm, out_hbm, idx_v, rows_v, out_v, sem):
        wid = lax.axis_index("s") * NC + lax.axis_index("c")
        base0 = wid * ppw

        pltpu.sync_copy(idx_hbm.at[pl.ds(base0 * _K, _PCH * _K)], idx_v.at[0])
        pltpu.async_copy(tbl_hbm.at[idx_v.at[0]], rows_v.at[0], sem.at[0])

        def step(it, carry):
            p = lax.rem(it, 2)
            q = 1 - p

            @pl.when(it + 1 < nit)
            def _():
                base_n = base0 + (it + 1) * _PCH
                pltpu.sync_copy(idx_hbm.at[pl.ds(base_n * _K, _PCH * _K)], idx_v.at[q])
                pltpu.async_copy(tbl_hbm.at[idx_v.at[q]], rows_v.at[q], sem.at[q])

            pltpu.make_async_copy(tbl_hbm.at[idx_v.at[p]], rows_v.at[p], sem.at[p]).wait()

            himask = jnp.full((16,), -65536, jnp.int32)
            sh16 = jnp.full((16,), 16, jnp.int32)

            def point(pt, c2):
                r0 = pt * _K
                for c in range(C2 // 16):
                    if packed:
                        w0 = rows_v[p, r0, pl.ds(c * 16, 16)]
                        acc_h = w0 & himask
                        acc_l = jax.lax.shift_left(w0, sh16)
                        for kk in range(1, _K):
                            w = rows_v[p, r0 + kk, pl.ds(c * 16, 16)]
                            acc_h = jnp.maximum(acc_h, w & himask)
                            acc_l = jnp.maximum(acc_l, jax.lax.shift_left(w, sh16))
                        out_v[pt, pl.ds(c * 16, 16)] = (
                            acc_h | jax.lax.shift_right_logical(acc_l, sh16))
                    else:
                        acc = rows_v[p, r0, pl.ds(c * 16, 16)]
                        for kk in range(1, _K):
                            acc = jnp.maximum(acc, rows_v[p, r0 + kk, pl.ds(c * 16, 16)])
                        out_v[pt, pl.ds(c * 16, 16)] = acc
                return c2

            lax.fori_loop(0, _PCH, point, 0)
            pltpu.sync_copy(out_v, out_hbm.at[pl.ds(base0 + it * _PCH, _PCH)])
            return carry

        lax.fori_loop(0, nit, step, 0)

    return pool


def _res_w2_body(x_ref, Wg1_ref, bg1_ref, W2_ref, b2_ref, o_ref):
    x = x_ref[...].astype(jnp.float32)
    g1 = jax.lax.dot(x, Wg1_ref[...], preferred_element_type=jnp.float32) + bg1_ref[...] + x
    g1 = jnp.maximum(g1, 0.0)
    inter = jax.lax.dot(g1, W2_ref[...], preferred_element_type=jnp.float32) + b2_ref[...]
    o_ref[...] = jnp.maximum(inter, 0.0).astype(jnp.bfloat16)


def _run_res_w2(x, Wg1, bg1, W2, b2):
    M = x.shape[0]
    return pl.pallas_call(
        _res_w2_body,
        grid=(M // _RD,),
        in_specs=[
            pl.BlockSpec((_RD, 128), lambda i: (i, 0)),
            pl.BlockSpec((128, 128), lambda i: (0, 0)),
            pl.BlockSpec((1, 128), lambda i: (0, 0)),
            pl.BlockSpec((128, 256), lambda i: (0, 0)),
            pl.BlockSpec((1, 256), lambda i: (0, 0)),
        ],
        out_specs=pl.BlockSpec((_RD, 256), lambda i: (i, 0)),
        out_shape=jax.ShapeDtypeStruct((M, 256), jnp.bfloat16),
    )(x, Wg1, bg1, W2, b2)


def _res_w3_max_body(x_ref, Wg2_ref, bg2_ref, W3_ref, b3_ref, o_ref):
    x = x_ref[0].astype(jnp.float32)
    g2 = jax.lax.dot(x, Wg2_ref[...], preferred_element_type=jnp.float32) + bg2_ref[...] + x
    g2 = jnp.maximum(g2, 0.0)
    graph = jax.lax.dot(g2, W3_ref[...], preferred_element_type=jnp.float32) + b3_ref[...]
    pm = jnp.max(graph, axis=0, keepdims=True)
    j = pl.program_id(1)

    @pl.when(j == 0)
    def _():
        o_ref[0] = pm

    @pl.when(j > 0)
    def _():
        o_ref[0] = jnp.maximum(o_ref[0], pm)


def _run_res_w3_max(x, Wg2, bg2, W3, b3):
    B, N, _ = x.shape
    return pl.pallas_call(
        _res_w3_max_body,
        grid=(B, N // _RD),
        in_specs=[
            pl.BlockSpec((1, _RD, 256), lambda b, n: (b, n, 0)),
            pl.BlockSpec((256, 256), lambda b, n: (0, 0)),
            pl.BlockSpec((1, 256), lambda b, n: (0, 0)),
            pl.BlockSpec((256, 512), lambda b, n: (0, 0)),
            pl.BlockSpec((1, 512), lambda b, n: (0, 0)),
        ],
        out_specs=pl.BlockSpec((1, 1, 512), lambda b, n: (b, 0, 0)),
        out_shape=jax.ShapeDtypeStruct((B, 1, 512), jnp.float32),
    )(x, Wg2, bg2, W3, b3)


def _gmlp_body(g_ref, W4_ref, b4_ref, W5_ref, b5_ref, o_ref):
    g = g_ref[...]
    h = jax.lax.dot(g, W4_ref[...], preferred_element_type=jnp.float32) + b4_ref[...]
    h = jnp.maximum(h, 0.0)
    o_ref[...] = jax.lax.dot(h, W5_ref[...], preferred_element_type=jnp.float32) + b5_ref[...]


def _run_gmlp(g, W4, b4, W5, b5):
    B = g.shape[0]
    F2 = W4.shape[1]
    return pl.pallas_call(
        _gmlp_body,
        out_shape=jax.ShapeDtypeStruct((B, F2), jnp.float32),
    )(g, W4, b4, W5, b5)


def kernel(pts, W1, b1, Wr1, br1, Wr2, br2, Wg1, bg1, W2, b2, Wg2, bg2, W3, b3, W4, b4, W5, b5):
    B, N, _ = pts.shape
    B2 = B // 2
    M2 = B2 * N
    pts_pad = jnp.pad(pts, ((0, 0), (0, 0), (0, 125)))
    pts_padT = jnp.transpose(pts_pad, (0, 2, 1))

    def as_i32(x, C):
        return jax.lax.bitcast_convert_type(x.reshape(M2, C // 2, 2), jnp.int32)

    def as_bf16(x, C):
        return jax.lax.bitcast_convert_type(x, jnp.bfloat16).reshape(M2, C)

    def half(sl):
        lf, idxg = _run_knn_mlp1(pts_pad[sl], pts_padT[sl], W1, b1.reshape(1, -1),
                                 Wr1, br1.reshape(1, -1), Wr2, br2.reshape(1, -1))
        idx_flat = idxg.reshape(M2 * _K)
        p1 = _make_pool(M2, 128, False)(lf.reshape(M2, 128), idx_flat)
        inter = _run_res_w2(p1, Wg1, bg1.reshape(1, -1), W2, b2.reshape(1, -1))
        p2 = as_bf16(_make_pool(M2, 256, True)(as_i32(inter, 256), idx_flat), 256)
        gmax = _run_res_w3_max(p2.reshape(B2, N, 256), Wg2, bg2.reshape(1, -1),
                               W3, b3.reshape(1, -1))
        return lf, gmax

    lfA, gA = half(slice(0, B2))
    lfB, gB = half(slice(B2, B))
    lf = jnp.concatenate([lfA, lfB], axis=0)
    gmax = jnp.concatenate([gA, gB], axis=0).reshape(B, -1)
    g = _run_gmlp(gmax, W4, b4.reshape(1, -1), W5, b5.reshape(1, -1))
    return (g.reshape(B, 1, -1), lf)


# speedup vs baseline: 1.1316x; 1.1316x over previous
import functools

import jax
import jax.numpy as jnp
from jax import lax
from jax.experimental import pallas as pl
from jax.experimental.pallas import tpu as pltpu
from jax.experimental.pallas import tpu_sc as plsc

_K = 16
_RA = 256
_RD = 512
_PCH = 8


def _knn_mlp1_body(xblk_ref, xT_ref, xfull_ref, W1_ref, b1_ref, Wr1_ref,
                   br1_ref, Wr2_ref, br2_ref, lf_ref, idx_ref):
    b = pl.program_id(0)
    N = xT_ref.shape[2]
    xblk = xblk_ref[0]
    xT = xT_ref[0]
    G = jax.lax.dot(xblk, xT, preferred_element_type=jnp.float32)
    xx_i = jnp.sum(xblk * xblk, axis=1, keepdims=True)
    xx_j = jnp.sum(xT * xT, axis=0, keepdims=True)
    d = 2.0 * G - xx_i - xx_j
    colf = jax.lax.broadcasted_iota(jnp.int32, (_RA, N), 1).astype(jnp.float32)
    idxs = []
    for k in range(_K):
        m = jnp.max(d, axis=1, keepdims=True)
        eq = d == m
        ik = jnp.min(jnp.where(eq, colf, 2.0 * N), axis=1, keepdims=True)
        idxs.append(ik)
        if k + 1 < _K:
            d = jnp.where(eq, -jnp.inf, d)
    idx = jnp.concatenate(idxs, axis=1).astype(jnp.int32)
    xfull = xfull_ref[0]
    oh0 = (colf == idxs[0]).astype(jnp.float32)
    oh1 = (colf == idxs[1]).astype(jnp.float32)
    nb0 = jax.lax.dot(oh0, xfull, preferred_element_type=jnp.float32)
    nb1 = jax.lax.dot(oh1, xfull, preferred_element_type=jnp.float32)
    acc = jnp.zeros((_RA, 128), jnp.float32) + b1_ref[...]
    for r in range(3):
        acc += xblk[:, r:r + 1] * W1_ref[r:r + 1, :]
    for a in range(3):
        for c in range(3):
            acc += (nb0[:, a:a + 1] * nb1[:, c:c + 1]) * W1_ref[3 + 3 * a + c:4 + 3 * a + c, :]
    h1 = jnp.maximum(acc, 0.0)
    h2 = jax.lax.dot(h1, Wr1_ref[...], preferred_element_type=jnp.float32) + br1_ref[...] + h1
    h2 = jnp.maximum(h2, 0.0)
    lf = jax.lax.dot(h2, Wr2_ref[...], preferred_element_type=jnp.float32) + br2_ref[...] + h2
    lf_ref[0] = jnp.maximum(lf, 0.0)
    idx_ref[0] = idx + b * N


def _run_knn_mlp1(pts_pad, pts_padT, W1, b1, Wr1, br1, Wr2, br2):
    B, N, _ = pts_pad.shape
    grid = (B, N // _RA)
    return pl.pallas_call(
        _knn_mlp1_body,
        grid=grid,
        in_specs=[
            pl.BlockSpec((1, _RA, 128), lambda b, n: (b, n, 0)),
            pl.BlockSpec((1, 128, N), lambda b, n: (b, 0, 0)),
            pl.BlockSpec((1, N, 128), lambda b, n: (b, 0, 0)),
            pl.BlockSpec((12, 128), lambda b, n: (0, 0)),
            pl.BlockSpec((1, 128), lambda b, n: (0, 0)),
            pl.BlockSpec((128, 128), lambda b, n: (0, 0)),
            pl.BlockSpec((1, 128), lambda b, n: (0, 0)),
            pl.BlockSpec((128, 128), lambda b, n: (0, 0)),
            pl.BlockSpec((1, 128), lambda b, n: (0, 0)),
        ],
        out_specs=[
            pl.BlockSpec((1, _RA, 128), lambda b, n: (b, n, 0)),
            pl.BlockSpec((1, _RA, _K), lambda b, n: (b, n, 0)),
        ],
        out_shape=[
            jax.ShapeDtypeStruct((B, N, 128), jnp.float32),
            jax.ShapeDtypeStruct((B, N, _K), jnp.int32),
        ],
    )(pts_pad, pts_padT, pts_pad, W1, b1, Wr1, br1, Wr2, br2)


def _make_pool(M, C, packed):
    info = plsc.get_sparse_core_info()
    NC, NS = info.num_cores, info.num_subcores
    NW = NC * NS
    ppw = M // NW
    nit = ppw // _PCH
    mesh = plsc.VectorSubcoreMesh(core_axis_name="c", subcore_axis_name="s")

    C2 = C // 2 if packed else C
    dt = jnp.int32 if packed else jnp.float32

    @functools.partial(
        pl.kernel, mesh=mesh,
        out_type=jax.ShapeDtypeStruct((M, C2), dt),
        scratch_types=[
            pltpu.VMEM((2, _PCH * _K), jnp.int32),
            pltpu.VMEM((2, _PCH * _K, C2), dt),
            pltpu.VMEM((_PCH, C2), dt),
            pltpu.SemaphoreType.DMA((2,)),
        ],
    )
    def pool(tbl_hbm, idx_hbm, out_hbm, idx_v, rows_v, out_v, sem):
        wid = lax.axis_index("s") * NC + lax.axis_index("c")
        base0 = wid * ppw

        pltpu.sync_copy(idx_hbm.at[pl.ds(base0 * _K, _PCH * _K)], idx_v.at[0])
        pltpu.async_copy(tbl_hbm.at[idx_v.at[0]], rows_v.at[0], sem.at[0])

        def step(it, carry):
            p = lax.rem(it, 2)
            q = 1 - p

            @pl.when(it + 1 < nit)
            def _():
                base_n = base0 + (it + 1) * _PCH
                pltpu.sync_copy(idx_hbm.at[pl.ds(base_n * _K, _PCH * _K)], idx_v.at[q])
                pltpu.async_copy(tbl_hbm.at[idx_v.at[q]], rows_v.at[q], sem.at[q])

            pltpu.make_async_copy(tbl_hbm.at[idx_v.at[p]], rows_v.at[p], sem.at[p]).wait()

            himask = jnp.full((16,), -65536, jnp.int32)
            sh16 = jnp.full((16,), 16, jnp.int32)

            def point(pt, c2):
                r0 = pt * _K
                for c in range(C2 // 16):
                    if packed:
                        w0 = rows_v[p, r0, pl.ds(c * 16, 16)]
                        acc_h = w0 & himask
                        acc_l = jax.lax.shift_left(w0, sh16)
                        for kk in range(1, _K):
                            w = rows_v[p, r0 + kk, pl.ds(c * 16, 16)]
                            acc_h = jnp.maximum(acc_h, w & himask)
                            acc_l = jnp.maximum(acc_l, jax.lax.shift_left(w, sh16))
                        out_v[pt, pl.ds(c * 16, 16)] = (
                            acc_h | jax.lax.shift_right_logical(acc_l, sh16))
                    else:
                        acc = rows_v[p, r0, pl.ds(c * 16, 16)]
                        for kk in range(1, _K):
                            acc = jnp.maximum(acc, rows_v[p, r0 + kk, pl.ds(c * 16, 16)])
                        out_v[pt, pl.ds(c * 16, 16)] = acc
                return c2

            lax.fori_loop(0, _PCH, point, 0)
            pltpu.sync_copy(out_v, out_hbm.at[pl.ds(base0 + it * _PCH, _PCH)])
            return carry

        lax.fori_loop(0, nit, step, 0)

    return pool


def _res_w2_body(x_ref, Wg1_ref, bg1_ref, W2_ref, b2_ref, o_ref):
    x = x_ref[...].astype(jnp.float32)
    g1 = jax.lax.dot(x, Wg1_ref[...], preferred_element_type=jnp.float32) + bg1_ref[...] + x
    g1 = jnp.maximum(g1, 0.0)
    inter = jax.lax.dot(g1, W2_ref[...], preferred_element_type=jnp.float32) + b2_ref[...]
    o_ref[...] = jnp.maximum(inter, 0.0)


def _run_res_w2(x, Wg1, bg1, W2, b2):
    M = x.shape[0]
    return pl.pallas_call(
        _res_w2_body,
        grid=(M // _RD,),
        in_specs=[
            pl.BlockSpec((_RD, 128), lambda i: (i, 0)),
            pl.BlockSpec((128, 128), lambda i: (0, 0)),
            pl.BlockSpec((1, 128), lambda i: (0, 0)),
            pl.BlockSpec((128, 256), lambda i: (0, 0)),
            pl.BlockSpec((1, 256), lambda i: (0, 0)),
        ],
        out_specs=pl.BlockSpec((_RD, 256), lambda i: (i, 0)),
        out_shape=jax.ShapeDtypeStruct((M, 256), jnp.float32),
    )(x, Wg1, bg1, W2, b2)


def _res_w3_max_body(x_ref, Wg2_ref, bg2_ref, W3_ref, b3_ref, o_ref):
    x = x_ref[0].astype(jnp.float32)
    g2 = jax.lax.dot(x, Wg2_ref[...], preferred_element_type=jnp.float32) + bg2_ref[...] + x
    g2 = jnp.maximum(g2, 0.0)
    graph = jax.lax.dot(g2, W3_ref[...], preferred_element_type=jnp.float32) + b3_ref[...]
    pm = jnp.max(graph, axis=0, keepdims=True)
    j = pl.program_id(1)

    @pl.when(j == 0)
    def _():
        o_ref[0] = pm

    @pl.when(j > 0)
    def _():
        o_ref[0] = jnp.maximum(o_ref[0], pm)


def _run_res_w3_max(x, Wg2, bg2, W3, b3):
    B, N, _ = x.shape
    return pl.pallas_call(
        _res_w3_max_body,
        grid=(B, N // _RD),
        in_specs=[
            pl.BlockSpec((1, _RD, 256), lambda b, n: (b, n, 0)),
            pl.BlockSpec((256, 256), lambda b, n: (0, 0)),
            pl.BlockSpec((1, 256), lambda b, n: (0, 0)),
            pl.BlockSpec((256, 512), lambda b, n: (0, 0)),
            pl.BlockSpec((1, 512), lambda b, n: (0, 0)),
        ],
        out_specs=pl.BlockSpec((1, 1, 512), lambda b, n: (b, 0, 0)),
        out_shape=jax.ShapeDtypeStruct((B, 1, 512), jnp.float32),
    )(x, Wg2, bg2, W3, b3)


def _gmlp_body(g_ref, W4_ref, b4_ref, W5_ref, b5_ref, o_ref):
    g = g_ref[...]
    h = jax.lax.dot(g, W4_ref[...], preferred_element_type=jnp.float32) + b4_ref[...]
    h = jnp.maximum(h, 0.0)
    o_ref[...] = jax.lax.dot(h, W5_ref[...], preferred_element_type=jnp.float32) + b5_ref[...]


def _run_gmlp(g, W4, b4, W5, b5):
    B = g.shape[0]
    F2 = W4.shape[1]
    return pl.pallas_call(
        _gmlp_body,
        out_shape=jax.ShapeDtypeStruct((B, F2), jnp.float32),
    )(g, W4, b4, W5, b5)


def kernel(pts, W1, b1, Wr1, br1, Wr2, br2, Wg1, bg1, W2, b2, Wg2, bg2, W3, b3, W4, b4, W5, b5):
    B, N, _ = pts.shape
    B2 = B // 2
    M2 = B2 * N
    pts_pad = jnp.pad(pts, ((0, 0), (0, 0), (0, 125)))
    pts_padT = jnp.transpose(pts_pad, (0, 2, 1))

    def as_i32(x, C):
        return jax.lax.bitcast_convert_type(x.reshape(M2, C // 2, 2), jnp.int32)

    def as_bf16(x, C):
        return jax.lax.bitcast_convert_type(x, jnp.bfloat16).reshape(M2, C)

    def half(sl):
        lf, idxg = _run_knn_mlp1(pts_pad[sl], pts_padT[sl], W1, b1.reshape(1, -1),
                                 Wr1, br1.reshape(1, -1), Wr2, br2.reshape(1, -1))
        idx_flat = idxg.reshape(M2 * _K)
        p1 = _make_pool(M2, 128, False)(lf.reshape(M2, 128), idx_flat)
        inter = _run_res_w2(p1, Wg1, bg1.reshape(1, -1), W2, b2.reshape(1, -1))
        p2 = _make_pool(M2, 256, False)(inter, idx_flat)
        gmax = _run_res_w3_max(p2.reshape(B2, N, 256), Wg2, bg2.reshape(1, -1),
                               W3, b3.reshape(1, -1))
        return lf, gmax

    lfA, gA = half(slice(0, B2))
    lfB, gB = half(slice(B2, B))
    lf = jnp.concatenate([lfA, lfB], axis=0)
    gmax = jnp.concatenate([gA, gB], axis=0).reshape(B, -1)
    g = _run_gmlp(gmax, W4, b4.reshape(1, -1), W5, b5.reshape(1, -1))
    return (g.reshape(B, 1, -1), lf)


# speedup vs baseline: 1.1329x; 1.0012x over previous
import functools

import jax
import jax.numpy as jnp
from jax import lax
from jax.experimental import pallas as pl
from jax.experimental.pallas import tpu as pltpu
from jax.experimental.pallas import tpu_sc as plsc

_K = 16
_RA = 256
_RD = 512
_PCH = 8


def _knn_mlp1_body(xblk_ref, xT_ref, xfull_ref, W1_ref, b1_ref, Wr1_ref,
                   br1_ref, Wr2_ref, br2_ref, lf_ref, idx_ref):
    b = pl.program_id(0)
    N = xT_ref.shape[2]
    xblk = xblk_ref[0]
    xT = xT_ref[0]
    G = jax.lax.dot(xblk, xT, preferred_element_type=jnp.float32)
    xx_i = jnp.sum(xblk * xblk, axis=1, keepdims=True)
    xx_j = jnp.sum(xT * xT, axis=0, keepdims=True)
    d = 2.0 * G - xx_i - xx_j
    colf = jax.lax.broadcasted_iota(jnp.int32, (_RA, N), 1).astype(jnp.float32)
    idxs = []
    for k in range(_K):
        m = jnp.max(d, axis=1, keepdims=True)
        eq = d == m
        ik = jnp.min(jnp.where(eq, colf, 2.0 * N), axis=1, keepdims=True)
        idxs.append(ik)
        if k + 1 < _K:
            d = jnp.where(eq, -jnp.inf, d)
    idx = jnp.concatenate(idxs, axis=1).astype(jnp.int32)
    xfull = xfull_ref[0]
    oh0 = (colf == idxs[0]).astype(jnp.float32)
    oh1 = (colf == idxs[1]).astype(jnp.float32)
    nb0 = jax.lax.dot(oh0, xfull, preferred_element_type=jnp.float32)
    nb1 = jax.lax.dot(oh1, xfull, preferred_element_type=jnp.float32)
    acc = jnp.zeros((_RA, 128), jnp.float32) + b1_ref[...]
    for r in range(3):
        acc += xblk[:, r:r + 1] * W1_ref[r:r + 1, :]
    for a in range(3):
        for c in range(3):
            acc += (nb0[:, a:a + 1] * nb1[:, c:c + 1]) * W1_ref[3 + 3 * a + c:4 + 3 * a + c, :]
    h1 = jnp.maximum(acc, 0.0)
    h2 = jax.lax.dot(h1, Wr1_ref[...], preferred_element_type=jnp.float32) + br1_ref[...] + h1
    h2 = jnp.maximum(h2, 0.0)
    lf = jax.lax.dot(h2, Wr2_ref[...], preferred_element_type=jnp.float32) + br2_ref[...] + h2
    lf_ref[0] = jnp.maximum(lf, 0.0)
    idx_ref[0] = idx + b * N


def _run_knn_mlp1(pts_pad, pts_padT, W1, b1, Wr1, br1, Wr2, br2):
    B, N, _ = pts_pad.shape
    grid = (B, N // _RA)
    return pl.pallas_call(
        _knn_mlp1_body,
        grid=grid,
        in_specs=[
            pl.BlockSpec((1, _RA, 128), lambda b, n: (b, n, 0)),
            pl.BlockSpec((1, 128, N), lambda b, n: (b, 0, 0)),
            pl.BlockSpec((1, N, 128), lambda b, n: (b, 0, 0)),
            pl.BlockSpec((12, 128), lambda b, n: (0, 0)),
            pl.BlockSpec((1, 128), lambda b, n: (0, 0)),
            pl.BlockSpec((128, 128), lambda b, n: (0, 0)),
            pl.BlockSpec((1, 128), lambda b, n: (0, 0)),
            pl.BlockSpec((128, 128), lambda b, n: (0, 0)),
            pl.BlockSpec((1, 128), lambda b, n: (0, 0)),
        ],
        out_specs=[
            pl.BlockSpec((1, _RA, 128), lambda b, n: (b, n, 0)),
            pl.BlockSpec((1, _RA, _K), lambda b, n: (b, n, 0)),
        ],
        out_shape=[
            jax.ShapeDtypeStruct((B, N, 128), jnp.float32),
            jax.ShapeDtypeStruct((B, N, _K), jnp.int32),
        ],
    )(pts_pad, pts_padT, pts_pad, W1, b1, Wr1, br1, Wr2, br2)


def _make_pool(M, C, packed):
    info = plsc.get_sparse_core_info()
    NC, NS = info.num_cores, info.num_subcores
    NW = NC * NS
    ppw = M // NW
    nit = ppw // _PCH
    mesh = plsc.VectorSubcoreMesh(core_axis_name="c", subcore_axis_name="s")

    C2 = C // 2 if packed else C
    dt = jnp.int32 if packed else jnp.float32

    @functools.partial(
        pl.kernel, mesh=mesh,
        out_type=jax.ShapeDtypeStruct((M, C2), dt),
        scratch_types=[
            pltpu.VMEM((2, _PCH * _K), jnp.int32),
            pltpu.VMEM((2, _PCH * _K, C2), dt),
            pltpu.VMEM((_PCH, C2), dt),
            pltpu.SemaphoreType.DMA((2,)),
        ],
    )
    def pool(tbl_hbm, idx_hbm, out_hbm, idx_v, rows_v, out_v, sem):
        wid = lax.axis_index("s") * NC + lax.axis_index("c")
        base0 = wid * ppw

        pltpu.sync_copy(idx_hbm.at[pl.ds(base0 * _K, _PCH * _K)], idx_v.at[0])
        pltpu.async_copy(tbl_hbm.at[idx_v.at[0]], rows_v.at[0], sem.at[0])

        def step(it, carry):
            p = lax.rem(it, 2)
            q = 1 - p

            @pl.when(it + 1 < nit)
            def _():
                base_n = base0 + (it + 1) * _PCH
                pltpu.sync_copy(idx_hbm.at[pl.ds(base_n * _K, _PCH * _K)], idx_v.at[q])
                pltpu.async_copy(tbl_hbm.at[idx_v.at[q]], rows_v.at[q], sem.at[q])

            pltpu.make_async_copy(tbl_hbm.at[idx_v.at[p]], rows_v.at[p], sem.at[p]).wait()

            himask = jnp.full((16,), -65536, jnp.int32)
            sh16 = jnp.full((16,), 16, jnp.int32)

            def point(pt, c2):
                r0 = pt * _K
                for c in range(C2 // 16):
                    if packed:
                        w0 = rows_v[p, r0, pl.ds(c * 16, 16)]
                        acc_h = w0 & himask
                        acc_l = jax.lax.shift_left(w0, sh16)
                        for kk in range(1, _K):
                            w = rows_v[p, r0 + kk, pl.ds(c * 16, 16)]
                            acc_h = jnp.maximum(acc_h, w & himask)
                            acc_l = jnp.maximum(acc_l, jax.lax.shift_left(w, sh16))
                        out_v[pt, pl.ds(c * 16, 16)] = (
                            acc_h | jax.lax.shift_right_logical(acc_l, sh16))
                    else:
                        acc = rows_v[p, r0, pl.ds(c * 16, 16)]
                        for kk in range(1, _K):
                            acc = jnp.maximum(acc, rows_v[p, r0 + kk, pl.ds(c * 16, 16)])
                        out_v[pt, pl.ds(c * 16, 16)] = acc
                return c2

            lax.fori_loop(0, _PCH, point, 0)
            pltpu.sync_copy(out_v, out_hbm.at[pl.ds(base0 + it * _PCH, _PCH)])
            return carry

        lax.fori_loop(0, nit, step, 0)

    return pool


def _res_w2_body(x_ref, Wg1_ref, bg1_ref, W2_ref, b2_ref, o_ref):
    x = x_ref[...].astype(jnp.float32)
    g1 = jax.lax.dot(x, Wg1_ref[...], preferred_element_type=jnp.float32) + bg1_ref[...] + x
    g1 = jnp.maximum(g1, 0.0)
    inter = jax.lax.dot(g1, W2_ref[...], preferred_element_type=jnp.float32) + b2_ref[...]
    o_ref[...] = jnp.maximum(inter, 0.0)


def _run_res_w2(x, Wg1, bg1, W2, b2):
    M = x.shape[0]
    return pl.pallas_call(
        _res_w2_body,
        grid=(M // _RD,),
        in_specs=[
            pl.BlockSpec((_RD, 128), lambda i: (i, 0)),
            pl.BlockSpec((128, 128), lambda i: (0, 0)),
            pl.BlockSpec((1, 128), lambda i: (0, 0)),
            pl.BlockSpec((128, 256), lambda i: (0, 0)),
            pl.BlockSpec((1, 256), lambda i: (0, 0)),
        ],
        out_specs=pl.BlockSpec((_RD, 256), lambda i: (i, 0)),
        out_shape=jax.ShapeDtypeStruct((M, 256), jnp.float32),
    )(x, Wg1, bg1, W2, b2)


def _res_w3_max_body(x_ref, Wg2_ref, bg2_ref, W3_ref, b3_ref, o_ref):
    x = x_ref[0].astype(jnp.float32)
    g2 = jax.lax.dot(x, Wg2_ref[...], preferred_element_type=jnp.float32) + bg2_ref[...] + x
    g2 = jnp.maximum(g2, 0.0)
    graph = jax.lax.dot(g2, W3_ref[...], preferred_element_type=jnp.float32) + b3_ref[...]
    pm = jnp.max(graph, axis=0, keepdims=True)
    j = pl.program_id(1)

    @pl.when(j == 0)
    def _():
        o_ref[0] = pm

    @pl.when(j > 0)
    def _():
        o_ref[0] = jnp.maximum(o_ref[0], pm)


def _run_res_w3_max(x, Wg2, bg2, W3, b3):
    B, N, _ = x.shape
    return pl.pallas_call(
        _res_w3_max_body,
        grid=(B, N // _RD),
        in_specs=[
            pl.BlockSpec((1, _RD, 256), lambda b, n: (b, n, 0)),
            pl.BlockSpec((256, 256), lambda b, n: (0, 0)),
            pl.BlockSpec((1, 256), lambda b, n: (0, 0)),
            pl.BlockSpec((256, 512), lambda b, n: (0, 0)),
            pl.BlockSpec((1, 512), lambda b, n: (0, 0)),
        ],
        out_specs=pl.BlockSpec((1, 1, 512), lambda b, n: (b, 0, 0)),
        out_shape=jax.ShapeDtypeStruct((B, 1, 512), jnp.float32),
    )(x, Wg2, bg2, W3, b3)


def _gmlp_body(g_ref, W4_ref, b4_ref, W5_ref, b5_ref, o_ref):
    g = g_ref[...]
    h = jax.lax.dot(g, W4_ref[...], preferred_element_type=jnp.float32) + b4_ref[...]
    h = jnp.maximum(h, 0.0)
    o_ref[...] = jax.lax.dot(h, W5_ref[...], preferred_element_type=jnp.float32) + b5_ref[...]


def _run_gmlp(g, W4, b4, W5, b5):
    B = g.shape[0]
    F2 = W4.shape[1]
    return pl.pallas_call(
        _gmlp_body,
        out_shape=jax.ShapeDtypeStruct((B, F2), jnp.float32),
    )(g, W4, b4, W5, b5)


def kernel(pts, W1, b1, Wr1, br1, Wr2, br2, Wg1, bg1, W2, b2, Wg2, bg2, W3, b3, W4, b4, W5, b5):
    B, N, _ = pts.shape
    B2 = B // 4
    M2 = B2 * N
    pts_pad = jnp.pad(pts, ((0, 0), (0, 0), (0, 125)))
    pts_padT = jnp.transpose(pts_pad, (0, 2, 1))

    def as_i32(x, C):
        return jax.lax.bitcast_convert_type(x.reshape(M2, C // 2, 2), jnp.int32)

    def as_bf16(x, C):
        return jax.lax.bitcast_convert_type(x, jnp.bfloat16).reshape(M2, C)

    def half(sl):
        lf, idxg = _run_knn_mlp1(pts_pad[sl], pts_padT[sl], W1, b1.reshape(1, -1),
                                 Wr1, br1.reshape(1, -1), Wr2, br2.reshape(1, -1))
        idx_flat = idxg.reshape(M2 * _K)
        p1 = _make_pool(M2, 128, False)(lf.reshape(M2, 128), idx_flat)
        inter = _run_res_w2(p1, Wg1, bg1.reshape(1, -1), W2, b2.reshape(1, -1))
        p2 = _make_pool(M2, 256, False)(inter, idx_flat)
        gmax = _run_res_w3_max(p2.reshape(B2, N, 256), Wg2, bg2.reshape(1, -1),
                               W3, b3.reshape(1, -1))
        return lf, gmax

    parts = [half(slice(i * B2, (i + 1) * B2)) for i in range(B // B2)]
    lf = jnp.concatenate([p[0] for p in parts], axis=0)
    gmax = jnp.concatenate([p[1] for p in parts], axis=0).reshape(B, -1)
    g = _run_gmlp(gmax, W4, b4.reshape(1, -1), W5, b5.reshape(1, -1))
    return (g.reshape(B, 1, -1), lf)


# speedup vs baseline: 1.1416x; 1.0077x over previous
import functools

import jax
import jax.numpy as jnp
from jax import lax
from jax.experimental import pallas as pl
from jax.experimental.pallas import tpu as pltpu
from jax.experimental.pallas import tpu_sc as plsc

_K = 16
_RA = 512
_RD = 512
_PCH = 8


def _knn_mlp1_body(xblk_ref, xT_ref, xfull_ref, W1_ref, b1_ref, Wr1_ref,
                   br1_ref, Wr2_ref, br2_ref, lf_ref, idx_ref):
    b = pl.program_id(0)
    N = xT_ref.shape[2]
    xblk = xblk_ref[0]
    xT = xT_ref[0]
    G = jax.lax.dot(xblk, xT, preferred_element_type=jnp.float32)
    xx_i = jnp.sum(xblk * xblk, axis=1, keepdims=True)
    xx_j = jnp.sum(xT * xT, axis=0, keepdims=True)
    d = 2.0 * G - xx_i - xx_j
    colf = jax.lax.broadcasted_iota(jnp.int32, (_RA, N), 1).astype(jnp.float32)
    idxs = []
    for k in range(_K):
        m = jnp.max(d, axis=1, keepdims=True)
        eq = d == m
        ik = jnp.min(jnp.where(eq, colf, 2.0 * N), axis=1, keepdims=True)
        idxs.append(ik)
        if k + 1 < _K:
            d = jnp.where(eq, -jnp.inf, d)
    idx = jnp.concatenate(idxs, axis=1).astype(jnp.int32)
    xfull = xfull_ref[0]
    oh0 = (colf == idxs[0]).astype(jnp.float32)
    oh1 = (colf == idxs[1]).astype(jnp.float32)
    nb0 = jax.lax.dot(oh0, xfull, preferred_element_type=jnp.float32)
    nb1 = jax.lax.dot(oh1, xfull, preferred_element_type=jnp.float32)
    acc = jnp.zeros((_RA, 128), jnp.float32) + b1_ref[...]
    for r in range(3):
        acc += xblk[:, r:r + 1] * W1_ref[r:r + 1, :]
    for a in range(3):
        for c in range(3):
            acc += (nb0[:, a:a + 1] * nb1[:, c:c + 1]) * W1_ref[3 + 3 * a + c:4 + 3 * a + c, :]
    h1 = jnp.maximum(acc, 0.0)
    h2 = jax.lax.dot(h1, Wr1_ref[...], preferred_element_type=jnp.float32) + br1_ref[...] + h1
    h2 = jnp.maximum(h2, 0.0)
    lf = jax.lax.dot(h2, Wr2_ref[...], preferred_element_type=jnp.float32) + br2_ref[...] + h2
    lf_ref[0] = jnp.maximum(lf, 0.0)
    idx_ref[0] = idx + b * N


def _run_knn_mlp1(pts_pad, pts_padT, W1, b1, Wr1, br1, Wr2, br2):
    B, N, _ = pts_pad.shape
    grid = (B, N // _RA)
    return pl.pallas_call(
        _knn_mlp1_body,
        grid=grid,
        in_specs=[
            pl.BlockSpec((1, _RA, 128), lambda b, n: (b, n, 0)),
            pl.BlockSpec((1, 128, N), lambda b, n: (b, 0, 0)),
            pl.BlockSpec((1, N, 128), lambda b, n: (b, 0, 0)),
            pl.BlockSpec((12, 128), lambda b, n: (0, 0)),
            pl.BlockSpec((1, 128), lambda b, n: (0, 0)),
            pl.BlockSpec((128, 128), lambda b, n: (0, 0)),
            pl.BlockSpec((1, 128), lambda b, n: (0, 0)),
            pl.BlockSpec((128, 128), lambda b, n: (0, 0)),
            pl.BlockSpec((1, 128), lambda b, n: (0, 0)),
        ],
        out_specs=[
            pl.BlockSpec((1, _RA, 128), lambda b, n: (b, n, 0)),
            pl.BlockSpec((1, _RA, _K), lambda b, n: (b, n, 0)),
        ],
        out_shape=[
            jax.ShapeDtypeStruct((B, N, 128), jnp.float32),
            jax.ShapeDtypeStruct((B, N, _K), jnp.int32),
        ],
    )(pts_pad, pts_padT, pts_pad, W1, b1, Wr1, br1, Wr2, br2)


def _make_pool(M, C, packed):
    info = plsc.get_sparse_core_info()
    NC, NS = info.num_cores, info.num_subcores
    NW = NC * NS
    ppw = M // NW
    nit = ppw // _PCH
    mesh = plsc.VectorSubcoreMesh(core_axis_name="c", subcore_axis_name="s")

    C2 = C // 2 if packed else C
    dt = jnp.int32 if packed else jnp.float32

    @functools.partial(
        pl.kernel, mesh=mesh,
        out_type=jax.ShapeDtypeStruct((M, C2), dt),
        scratch_types=[
            pltpu.VMEM((2, _PCH * _K), jnp.int32),
            pltpu.VMEM((2, _PCH * _K, C2), dt),
            pltpu.VMEM((_PCH, C2), dt),
            pltpu.SemaphoreType.DMA((2,)),
        ],
    )
    def pool(tbl_hbm, idx_hbm, out_hbm, idx_v, rows_v, out_v, sem):
        wid = lax.axis_index("s") * NC + lax.axis_index("c")
        base0 = wid * ppw

        pltpu.sync_copy(idx_hbm.at[pl.ds(base0 * _K, _PCH * _K)], idx_v.at[0])
        pltpu.async_copy(tbl_hbm.at[idx_v.at[0]], rows_v.at[0], sem.at[0])

        def step(it, carry):
            p = lax.rem(it, 2)
            q = 1 - p

            @pl.when(it + 1 < nit)
            def _():
                base_n = base0 + (it + 1) * _PCH
                pltpu.sync_copy(idx_hbm.at[pl.ds(base_n * _K, _PCH * _K)], idx_v.at[q])
                pltpu.async_copy(tbl_hbm.at[idx_v.at[q]], rows_v.at[q], sem.at[q])

            pltpu.make_async_copy(tbl_hbm.at[idx_v.at[p]], rows_v.at[p], sem.at[p]).wait()

            himask = jnp.full((16,), -65536, jnp.int32)
            sh16 = jnp.full((16,), 16, jnp.int32)

            def point(pt, c2):
                r0 = pt * _K
                for c in range(C2 // 16):
                    if packed:
                        w0 = rows_v[p, r0, pl.ds(c * 16, 16)]
                        acc_h = w0 & himask
                        acc_l = jax.lax.shift_left(w0, sh16)
                        for kk in range(1, _K):
                            w = rows_v[p, r0 + kk, pl.ds(c * 16, 16)]
                            acc_h = jnp.maximum(acc_h, w & himask)
                            acc_l = jnp.maximum(acc_l, jax.lax.shift_left(w, sh16))
                        out_v[pt, pl.ds(c * 16, 16)] = (
                            acc_h | jax.lax.shift_right_logical(acc_l, sh16))
                    else:
                        acc = rows_v[p, r0, pl.ds(c * 16, 16)]
                        for kk in range(1, _K):
                            acc = jnp.maximum(acc, rows_v[p, r0 + kk, pl.ds(c * 16, 16)])
                        out_v[pt, pl.ds(c * 16, 16)] = acc
                return c2

            lax.fori_loop(0, _PCH, point, 0)
            pltpu.sync_copy(out_v, out_hbm.at[pl.ds(base0 + it * _PCH, _PCH)])
            return carry

        lax.fori_loop(0, nit, step, 0)

    return pool


def _res_w2_body(x_ref, Wg1_ref, bg1_ref, W2_ref, b2_ref, o_ref):
    x = x_ref[...].astype(jnp.float32)
    g1 = jax.lax.dot(x, Wg1_ref[...], preferred_element_type=jnp.float32) + bg1_ref[...] + x
    g1 = jnp.maximum(g1, 0.0)
    inter = jax.lax.dot(g1, W2_ref[...], preferred_element_type=jnp.float32) + b2_ref[...]
    o_ref[...] = jnp.maximum(inter, 0.0)


def _run_res_w2(x, Wg1, bg1, W2, b2):
    M = x.shape[0]
    return pl.pallas_call(
        _res_w2_body,
        grid=(M // _RD,),
        in_specs=[
            pl.BlockSpec((_RD, 128), lambda i: (i, 0)),
            pl.BlockSpec((128, 128), lambda i: (0, 0)),
            pl.BlockSpec((1, 128), lambda i: (0, 0)),
            pl.BlockSpec((128, 256), lambda i: (0, 0)),
            pl.BlockSpec((1, 256), lambda i: (0, 0)),
        ],
        out_specs=pl.BlockSpec((_RD, 256), lambda i: (i, 0)),
        out_shape=jax.ShapeDtypeStruct((M, 256), jnp.float32),
    )(x, Wg1, bg1, W2, b2)


def _res_w3_max_body(x_ref, Wg2_ref, bg2_ref, W3_ref, b3_ref, o_ref):
    x = x_ref[0].astype(jnp.float32)
    g2 = jax.lax.dot(x, Wg2_ref[...], preferred_element_type=jnp.float32) + bg2_ref[...] + x
    g2 = jnp.maximum(g2, 0.0)
    graph = jax.lax.dot(g2, W3_ref[...], preferred_element_type=jnp.float32) + b3_ref[...]
    pm = jnp.max(graph, axis=0, keepdims=True)
    j = pl.program_id(1)

    @pl.when(j == 0)
    def _():
        o_ref[0] = pm

    @pl.when(j > 0)
    def _():
        o_ref[0] = jnp.maximum(o_ref[0], pm)


def _run_res_w3_max(x, Wg2, bg2, W3, b3):
    B, N, _ = x.shape
    return pl.pallas_call(
        _res_w3_max_body,
        grid=(B, N // _RD),
        in_specs=[
            pl.BlockSpec((1, _RD, 256), lambda b, n: (b, n, 0)),
            pl.BlockSpec((256, 256), lambda b, n: (0, 0)),
            pl.BlockSpec((1, 256), lambda b, n: (0, 0)),
            pl.BlockSpec((256, 512), lambda b, n: (0, 0)),
            pl.BlockSpec((1, 512), lambda b, n: (0, 0)),
        ],
        out_specs=pl.BlockSpec((1, 1, 512), lambda b, n: (b, 0, 0)),
        out_shape=jax.ShapeDtypeStruct((B, 1, 512), jnp.float32),
    )(x, Wg2, bg2, W3, b3)


def _gmlp_body(g_ref, W4_ref, b4_ref, W5_ref, b5_ref, o_ref):
    g = g_ref[...]
    h = jax.lax.dot(g, W4_ref[...], preferred_element_type=jnp.float32) + b4_ref[...]
    h = jnp.maximum(h, 0.0)
    o_ref[...] = jax.lax.dot(h, W5_ref[...], preferred_element_type=jnp.float32) + b5_ref[...]


def _run_gmlp(g, W4, b4, W5, b5):
    B = g.shape[0]
    F2 = W4.shape[1]
    return pl.pallas_call(
        _gmlp_body,
        out_shape=jax.ShapeDtypeStruct((B, F2), jnp.float32),
    )(g, W4, b4, W5, b5)


def kernel(pts, W1, b1, Wr1, br1, Wr2, br2, Wg1, bg1, W2, b2, Wg2, bg2, W3, b3, W4, b4, W5, b5):
    B, N, _ = pts.shape
    B2 = B // 2
    M2 = B2 * N
    pts_pad = jnp.pad(pts, ((0, 0), (0, 0), (0, 125)))
    pts_padT = jnp.transpose(pts_pad, (0, 2, 1))

    def as_i32(x, C):
        return jax.lax.bitcast_convert_type(x.reshape(M2, C // 2, 2), jnp.int32)

    def as_bf16(x, C):
        return jax.lax.bitcast_convert_type(x, jnp.bfloat16).reshape(M2, C)

    def half(sl):
        lf, idxg = _run_knn_mlp1(pts_pad[sl], pts_padT[sl], W1, b1.reshape(1, -1),
                                 Wr1, br1.reshape(1, -1), Wr2, br2.reshape(1, -1))
        idx_flat = idxg.reshape(M2 * _K)
        p1 = _make_pool(M2, 128, False)(lf.reshape(M2, 128), idx_flat)
        inter = _run_res_w2(p1, Wg1, bg1.reshape(1, -1), W2, b2.reshape(1, -1))
        p2 = _make_pool(M2, 256, False)(inter, idx_flat)
        gmax = _run_res_w3_max(p2.reshape(B2, N, 256), Wg2, bg2.reshape(1, -1),
                               W3, b3.reshape(1, -1))
        return lf, gmax

    parts = [half(slice(i * B2, (i + 1) * B2)) for i in range(B // B2)]
    lf = jnp.concatenate([p[0] for p in parts], axis=0)
    gmax = jnp.concatenate([p[1] for p in parts], axis=0).reshape(B, -1)
    g = _run_gmlp(gmax, W4, b4.reshape(1, -1), W5, b5.reshape(1, -1))
    return (g.reshape(B, 1, -1), lf)
